# baseline (speedup 1.0000x reference)
import functools
import math

import jax
import jax.numpy as jnp
from jax import lax
from jax.experimental import pallas as pl
from jax.experimental.pallas import tpu as pltpu
from jax.experimental.pallas import tpu_sc as plsc

_N_ROI = 128
_K0 = 64
_K1 = 32
_NCL = 8
_G = 8


@functools.lru_cache(maxsize=None)
def _adj_builder(nc, ns, ch, n_a0):
    slice_sz = n_a0 // ns
    mesh = plsc.VectorSubcoreMesh(
        core_axis_name="c", subcore_axis_name="s",
        num_cores=nc, num_subcores=ns)

    @functools.partial(
        pl.kernel,
        out_type=jax.ShapeDtypeStruct((nc, n_a0), jnp.float32),
        mesh=mesh,
        scratch_types=[
            pltpu.VMEM((ch,), jnp.int32),
            pltpu.VMEM((ch,), jnp.int32),
            pltpu.VMEM((ch,), jnp.float32),
            pltpu.VMEM_SHARED((n_a0,), jnp.float32),
            pltpu.SemaphoreType.DMA,
            pltpu.SemaphoreType.DMA,
        ],
    )
    def adj(dst_hbm, src_hbm, ew_hbm, zero_hbm, out_hbm,
            dbuf, sbuf, updb, accum, zsem, sem):
        c = lax.axis_index("c")
        s = lax.axis_index("s")
        w = s * nc + c
        zcp = pltpu.async_copy(
            zero_hbm, accum.at[pl.ds(s * slice_sz, slice_sz)], zsem)
        ld = pltpu.async_copy(dst_hbm.at[w], dbuf, sem)
        ls = pltpu.async_copy(src_hbm.at[w], sbuf, sem)
        lw = pltpu.async_copy(ew_hbm.at[w], updb, sem)
        ld.wait()
        ls.wait()
        def _idx_body(i, carry):
            d = dbuf[pl.ds(i * 16, 16)]
            sv = sbuf[pl.ds(i * 16, 16)]
            dbuf[pl.ds(i * 16, 16)] = (
                lax.shift_left(d, 7) + jnp.bitwise_and(sv, 127))
            return carry

        lax.fori_loop(0, ch // 16, _idx_body, 0)
        lw.wait()
        zcp.wait()
        plsc.subcore_barrier()
        pltpu.async_copy(updb, accum.at[dbuf], sem, add=True).wait()
        plsc.subcore_barrier()
        pltpu.sync_copy(accum.at[pl.ds(s * slice_sz, slice_sz)],
                        out_hbm.at[c].at[pl.ds(s * slice_sz, slice_sz)])

    return adj


def _dot(a, b):
    return lax.dot_general(a, b, (((1,), (0,)), ((), ())),
                           preferred_element_type=jnp.float32)


def _dot00(a, b):
    return lax.dot_general(a, b, (((0,), (0,)), ((), ())),
                           preferred_element_type=jnp.float32)


def _sigmoid(z):
    e = jnp.exp(-jnp.abs(z))
    return jnp.where(z >= 0, 1.0 / (1.0 + e), e / (1.0 + e))


def _softmax_row(t):
    m = jnp.max(t, axis=1, keepdims=True)
    e = jnp.exp(t - m)
    return e / jnp.sum(e, axis=1, keepdims=True)


def _before_mat(s_col, s_row, tieb):
    m = jnp.where((s_row > s_col) | ((s_row == s_col) & tieb), 1.0, 0.0)
    return m.astype(jnp.bfloat16)


def _mk_e9_rmat(f1, f0):
    e_r = lax.broadcasted_iota(jnp.int32, (_NCL + 1, 288), 0)
    e_c = lax.broadcasted_iota(jnp.int32, (_NCL + 1, 288), 1)
    e9 = jnp.where(lax.shift_right_logical(e_c, 5) == e_r, f1, f0)
    r_r = lax.broadcasted_iota(jnp.int32, (288, 32), 0)
    r_c = lax.broadcasted_iota(jnp.int32, (288, 32), 1)
    rmat = jnp.where(jnp.bitwise_and(r_r, 31) == r_c, f1, f0)
    return e9, rmat


def _tca_body(x_ref, w10_ref, w2a0_ref, oxt_ref):
    n = _N_ROI
    f1 = jnp.float32(1.0)
    f0 = jnp.float32(0.0)
    e9, rmat = _mk_e9_rmat(f1, f0)
    h0e = jnp.concatenate(
        [jnp.maximum(w10_ref[...], 0.0), jnp.full((n, 1), f1)], axis=1)
    hg0 = _dot(h0e, e9)
    hg0t = jnp.concatenate([hg0] * _G, axis=0)
    y0all = _dot(x_ref[...].reshape(_G * n, n), w2a0_ref[...])
    oxt_ref[...] = _dot(hg0t * y0all, rmat)


def _tc_body(xt0_ref, a0_ref, b0_ref, w0_ref,
             w11_ref, w2a1_ref, b1_ref, w1_ref,
             m1w_ref, m1b_ref, m1a_ref, m1g_ref, m1be_ref,
             m2w_ref, m2b_ref, m2a_ref, m2g_ref, m2be_ref,
             m3w_ref, m3b_ref, out_ref):
    n = _N_ROI
    f1 = jnp.float32(1.0)
    f0 = jnp.float32(0.0)
    rw11 = jnp.maximum(w11_ref[...], 0.0)
    w2a1 = w2a1_ref[...]
    nrm0 = jnp.sqrt(jnp.sum(w0_ref[...] ** 2, axis=1, keepdims=True))
    nrm1 = jnp.sqrt(jnp.sum(w1_ref[...] ** 2, axis=1, keepdims=True))
    w0col = jnp.transpose(w0_ref[...]) / nrm0
    w1col = jnp.transpose(w1_ref[...]) / nrm1
    i0r = lax.broadcasted_iota(jnp.int32, (1, _K0), 1).astype(jnp.float32)
    i0c = lax.broadcasted_iota(jnp.int32, (_K0, 1), 0).astype(jnp.float32)
    i1r = lax.broadcasted_iota(jnp.int32, (1, _K1), 1).astype(jnp.float32)
    i1c = lax.broadcasted_iota(jnp.int32, (_K1, 1), 0).astype(jnp.float32)

    ones128 = jnp.full((n, 1), jnp.bfloat16(1))
    ones64 = jnp.full((_K0, 1), jnp.bfloat16(1))
    e9, rmat = _mk_e9_rmat(f1, f0)

    def _tieb(k):
        r_ = lax.broadcasted_iota(jnp.int32, (k, k), 0)
        c_ = lax.broadcasted_iota(jnp.int32, (k, k), 1)
        return c_ < r_

    tieb0, tieb1 = _tieb(n), _tieb(_K0)

    aall = a0_ref[0]
    for p in range(1, a0_ref.shape[0]):
        aall = aall + a0_ref[p]

    xt0all = xt0_ref[...]
    x1s = [jnp.maximum(_dot(aall[g], xt0all[g * n:(g + 1) * n])
                       + b0_ref[...], 0.0) for g in range(_G)]

    s0s = [_sigmoid(_dot(x1s[g], w0col)) for g in range(_G)]
    s0rows = jnp.transpose(jnp.concatenate(s0s, axis=1))
    ms = [_before_mat(s0s[g], s0rows[g:g + 1], tieb0) for g in range(_G)]
    r0cs = [_dot(ms[g], ones128) for g in range(_G)]
    r0rows = jnp.transpose(jnp.concatenate(r0cs, axis=1))
    p0ts = [jnp.where(i0c == r0rows[g:g + 1], f1, f0) for g in range(_G)]
    p0s = [jnp.where(r0cs[g] == i0r, f1, f0) for g in range(_G)]
    t0s = [_dot00(s0s[g], p0s[g]) for g in range(_G)]
    nx0s = [_dot(p0ts[g] * s0rows[g:g + 1], x1s[g]) for g in range(_G)]
    pas = [_dot(p0ts[g], aall[g]) for g in range(_G)]
    a1s = [_dot(pas[g], p0s[g]) for g in range(_G)]
    h1s = [_dot(p0ts[g], rw11) for g in range(_G)]

    nx0all = jnp.concatenate(nx0s, axis=0)
    h1e = jnp.concatenate(
        [jnp.concatenate(h1s, axis=0),
         jnp.full((_G * _K0, 1), f1)], axis=1)
    hg1 = _dot(h1e, e9)
    y1all = _dot(nx0all, w2a1)
    xt1all = _dot(hg1 * y1all, rmat)
    x2s = [jnp.maximum(_dot(a1s[g], xt1all[g * _K0:(g + 1) * _K0])
                       + b1_ref[...], 0.0) for g in range(_G)]

    s1s = [_sigmoid(_dot(x2s[g], w1col)) for g in range(_G)]
    s1rows = jnp.transpose(jnp.concatenate(s1s, axis=1))
    m1s = [_before_mat(s1s[g], s1rows[g:g + 1], tieb1) for g in range(_G)]
    r1cs = [_dot(m1s[g], ones64) for g in range(_G)]
    r1rows = jnp.transpose(jnp.concatenate(r1cs, axis=1))
    p1ts = [jnp.where(i1c == r1rows[g:g + 1], f1, f0) for g in range(_G)]
    p1s = [jnp.where(r1cs[g] == i1r, f1, f0) for g in range(_G)]
    t1s = [_dot00(s1s[g], p1s[g]) for g in range(_G)]
    nx1s = [_dot(p1ts[g] * s1rows[g:g + 1], x2s[g]) for g in range(_G)]

    rows = [jnp.concatenate(
        [jnp.mean(nx0s[g], axis=0, keepdims=True),
         jnp.max(nx0s[g], axis=0, keepdims=True),
         jnp.mean(nx1s[g], axis=0, keepdims=True),
         jnp.max(nx1s[g], axis=0, keepdims=True)], axis=1)
        for g in range(_G)]

    t0all = jnp.concatenate(t0s, axis=0)
    t1all = jnp.concatenate(t1s, axis=0)

    r = jnp.concatenate(rows, axis=0)
    c0 = 1.0 / math.sqrt(1.0 + 1e-5)
    h = _dot(r, m1w_ref[...]) + m1b_ref[...]
    h = jnp.where(h >= 0, h, m1a_ref[...] * h) * c0 * m1g_ref[...] + m1be_ref[...]
    h = _dot(h, m2w_ref[...]) + m2b_ref[...]
    h = jnp.where(h >= 0, h, m2a_ref[...] * h) * c0 * m2g_ref[...] + m2be_ref[...]
    logits = _dot(h, m3w_ref[...]) + m3b_ref[...]
    mx = jnp.max(logits, axis=1, keepdims=True)
    e = jnp.exp(logits - mx)
    out_ref[...] = jnp.concatenate(
        [logits - mx - jnp.log(jnp.sum(e, axis=1, keepdims=True)),
         _softmax_row(t0all), _softmax_row(t1all), t0all], axis=1)


def _gnn_tc(x3, a0, w10, w2a0, b0, w0, w11, w2a1, b1, w1,
            m1w, m1b, m1a, m1g, m1be, m2w, m2b, m2a, m2g, m2be, m3w, m3b):
    bsz = x3.shape[0]
    grid = (bsz // _G,)

    def full(shape):
        zeros = (0,) * len(shape)
        return pl.BlockSpec(shape, lambda b, z=zeros: z)

    xt0 = pl.pallas_call(
        _tca_body,
        grid=grid,
        in_specs=[
            pl.BlockSpec((_G, _N_ROI, _N_ROI), lambda b: (b, 0, 0)),
            full(w10.shape), full(w2a0.shape),
        ],
        out_specs=pl.BlockSpec((_G * _N_ROI, 32), lambda b: (b, 0)),
        out_shape=jax.ShapeDtypeStruct((bsz * _N_ROI, 32), jnp.float32),
    )(x3, w10, w2a0)

    nc = a0.shape[0]
    in_specs = [
        pl.BlockSpec((_G * _N_ROI, 32), lambda b: (b, 0)),
        pl.BlockSpec((nc, _G, _N_ROI, _N_ROI), lambda b: (0, b, 0, 0)),
        full(b0.shape), full(w0.shape),
        full(w11.shape), full(w2a1.shape), full(b1.shape), full(w1.shape),
        full(m1w.shape), full(m1b.shape), full(m1a.shape), full(m1g.shape),
        full(m1be.shape), full(m2w.shape), full(m2b.shape), full(m2a.shape),
        full(m2g.shape), full(m2be.shape), full(m3w.shape), full(m3b.shape),
    ]
    npk = 2 + _K0 + _K1 + _K0
    out = pl.pallas_call(
        _tc_body,
        grid=grid,
        in_specs=in_specs,
        out_specs=pl.BlockSpec((_G, npk), lambda b: (b, 0)),
        out_shape=jax.ShapeDtypeStruct((bsz, npk), jnp.float32),
    )(xt0, a0, b0, w0, w11, w2a1, b1, w1,
      m1w, m1b, m1a, m1g, m1be, m2w, m2b, m2a, m2g, m2be, m3w, m3b)
    return (out[:, :2], out[:, 2:2 + _K0],
            out[:, 2 + _K0:2 + _K0 + _K1], out[:, 2 + _K0 + _K1:])


def _aug_weights(w2, b2, in_c):
    w2r = w2.reshape(_NCL, in_c, 32).transpose(1, 0, 2).reshape(in_c, _NCL * 32)
    return jnp.concatenate([w2r, b2.reshape(in_c, 32)], axis=1)


def kernel(x, edge_index, edge_attr, pos, batch, params):
    nroi = _N_ROI
    bsz = x.shape[0] // nroi
    e = edge_attr.shape[0]
    info = plsc.get_sparse_core_info()
    ns = int(info.num_subcores)
    nc = 1
    nw = nc * ns
    ch = e // nw
    n_a0 = bsz * nroi * nroi

    src = edge_index[0]
    dst = edge_index[1]
    a0p = _adj_builder(nc, ns, ch, n_a0)(
        dst.reshape(nw, ch), src.reshape(nw, ch),
        edge_attr.reshape(nw, ch),
        jnp.zeros((n_a0 // ns,), jnp.float32))
    a0 = a0p.reshape(nc, bsz, nroi, nroi)

    p = params
    w2a0 = _aug_weights(p['conv0_W2'], p['conv0_b2'], nroi)
    w2a1 = _aug_weights(p['conv1_W2'], p['conv1_b2'], 32)
    xo, sn0, sn1, sc0 = _gnn_tc(
        x.reshape(bsz, nroi, x.shape[1]), a0,
        p['conv0_W1'], w2a0, p['conv0_bias'].reshape(1, 32),
        p['pool0_w'].reshape(1, 32),
        p['conv1_W1'], w2a1, p['conv1_bias'].reshape(1, 32),
        p['pool1_w'].reshape(1, 32),
        p['mlp1_W'], p['mlp1_b'].reshape(1, -1), p['mlp1_a'].reshape(1, 1),
        p['mlp1_gamma'].reshape(1, -1), p['mlp1_beta'].reshape(1, -1),
        p['mlp2_W'], p['mlp2_b'].reshape(1, -1), p['mlp2_a'].reshape(1, 1),
        p['mlp2_gamma'].reshape(1, -1), p['mlp2_beta'].reshape(1, -1),
        p['mlp3_W'], p['mlp3_b'].reshape(1, -1))
    return (xo, p['pool0_w'], p['pool1_w'], sn0, sn1, sc0)


# speedup vs baseline: 1.1313x; 1.1313x over previous
import functools
import math

import jax
import jax.numpy as jnp
from jax import lax
from jax.experimental import pallas as pl
from jax.experimental.pallas import tpu as pltpu
from jax.experimental.pallas import tpu_sc as plsc

_N_ROI = 128
_K0 = 64
_K1 = 32
_NCL = 8
_G = 16


@functools.lru_cache(maxsize=None)
def _adj_builder(nc, ns, ch, n_a0):
    slice_sz = n_a0 // ns
    mesh = plsc.VectorSubcoreMesh(
        core_axis_name="c", subcore_axis_name="s",
        num_cores=nc, num_subcores=ns)

    @functools.partial(
        pl.kernel,
        out_type=jax.ShapeDtypeStruct((nc, n_a0), jnp.float32),
        mesh=mesh,
        scratch_types=[
            pltpu.VMEM((ch,), jnp.int32),
            pltpu.VMEM((ch,), jnp.int32),
            pltpu.VMEM((ch,), jnp.float32),
            pltpu.VMEM_SHARED((n_a0,), jnp.float32),
            pltpu.SemaphoreType.DMA,
            pltpu.SemaphoreType.DMA,
        ],
    )
    def adj(dst_hbm, src_hbm, ew_hbm, zero_hbm, out_hbm,
            dbuf, sbuf, updb, accum, zsem, sem):
        c = lax.axis_index("c")
        s = lax.axis_index("s")
        w = s * nc + c
        zcp = pltpu.async_copy(
            zero_hbm, accum.at[pl.ds(s * slice_sz, slice_sz)], zsem)
        ld = pltpu.async_copy(dst_hbm.at[w], dbuf, sem)
        ls = pltpu.async_copy(src_hbm.at[w], sbuf, sem)
        lw = pltpu.async_copy(ew_hbm.at[w], updb, sem)
        ld.wait()
        ls.wait()
        def _idx_body(i, carry):
            d = dbuf[pl.ds(i * 16, 16)]
            sv = sbuf[pl.ds(i * 16, 16)]
            dbuf[pl.ds(i * 16, 16)] = (
                lax.shift_left(d, 7) + jnp.bitwise_and(sv, 127))
            return carry

        lax.fori_loop(0, ch // 16, _idx_body, 0)
        lw.wait()
        zcp.wait()
        plsc.subcore_barrier()
        pltpu.async_copy(updb, accum.at[dbuf], sem, add=True).wait()
        plsc.subcore_barrier()
        pltpu.sync_copy(accum.at[pl.ds(s * slice_sz, slice_sz)],
                        out_hbm.at[c].at[pl.ds(s * slice_sz, slice_sz)])

    return adj


def _dot(a, b):
    return lax.dot_general(a, b, (((1,), (0,)), ((), ())),
                           preferred_element_type=jnp.float32)


def _dot00(a, b):
    return lax.dot_general(a, b, (((0,), (0,)), ((), ())),
                           preferred_element_type=jnp.float32)


def _sigmoid(z):
    e = jnp.exp(-jnp.abs(z))
    return jnp.where(z >= 0, 1.0 / (1.0 + e), e / (1.0 + e))


def _softmax_row(t):
    m = jnp.max(t, axis=1, keepdims=True)
    e = jnp.exp(t - m)
    return e / jnp.sum(e, axis=1, keepdims=True)


def _before_mat(s_col, s_row, tieb):
    m = jnp.where((s_row > s_col) | ((s_row == s_col) & tieb), 1.0, 0.0)
    return m.astype(jnp.bfloat16)


def _mk_e9_rmat(f1, f0):
    e_r = lax.broadcasted_iota(jnp.int32, (_NCL + 1, 288), 0)
    e_c = lax.broadcasted_iota(jnp.int32, (_NCL + 1, 288), 1)
    e9 = jnp.where(lax.shift_right_logical(e_c, 5) == e_r, f1, f0)
    r_r = lax.broadcasted_iota(jnp.int32, (288, 32), 0)
    r_c = lax.broadcasted_iota(jnp.int32, (288, 32), 1)
    rmat = jnp.where(jnp.bitwise_and(r_r, 31) == r_c, f1, f0)
    return e9, rmat


def _tca_body(x_ref, w10_ref, w2a0_ref, oxt_ref):
    n = _N_ROI
    f1 = jnp.float32(1.0)
    f0 = jnp.float32(0.0)
    e9, rmat = _mk_e9_rmat(f1, f0)
    h0e = jnp.concatenate(
        [jnp.maximum(w10_ref[...], 0.0), jnp.full((n, 1), f1)], axis=1)
    hg0 = _dot(h0e, e9)
    hg0t = jnp.concatenate([hg0] * _G, axis=0)
    y0all = _dot(x_ref[...].reshape(_G * n, n), w2a0_ref[...])
    oxt_ref[...] = _dot(hg0t * y0all, rmat)


def _tc_body(xt0_ref, a0_ref, b0_ref, w0_ref,
             w11_ref, w2a1_ref, b1_ref, w1_ref,
             m1w_ref, m1b_ref, m1a_ref, m1g_ref, m1be_ref,
             m2w_ref, m2b_ref, m2a_ref, m2g_ref, m2be_ref,
             m3w_ref, m3b_ref, out_ref):
    n = _N_ROI
    f1 = jnp.float32(1.0)
    f0 = jnp.float32(0.0)
    rw11 = jnp.maximum(w11_ref[...], 0.0)
    w2a1 = w2a1_ref[...]
    nrm0 = jnp.sqrt(jnp.sum(w0_ref[...] ** 2, axis=1, keepdims=True))
    nrm1 = jnp.sqrt(jnp.sum(w1_ref[...] ** 2, axis=1, keepdims=True))
    w0col = jnp.transpose(w0_ref[...]) / nrm0
    w1col = jnp.transpose(w1_ref[...]) / nrm1
    i0r = lax.broadcasted_iota(jnp.int32, (1, _K0), 1).astype(jnp.float32)
    i0c = lax.broadcasted_iota(jnp.int32, (_K0, 1), 0).astype(jnp.float32)
    i1r = lax.broadcasted_iota(jnp.int32, (1, _K1), 1).astype(jnp.float32)
    i1c = lax.broadcasted_iota(jnp.int32, (_K1, 1), 0).astype(jnp.float32)

    ones128 = jnp.full((n, 1), jnp.bfloat16(1))
    ones64 = jnp.full((_K0, 1), jnp.bfloat16(1))
    e9, rmat = _mk_e9_rmat(f1, f0)

    def _tieb(k):
        r_ = lax.broadcasted_iota(jnp.int32, (k, k), 0)
        c_ = lax.broadcasted_iota(jnp.int32, (k, k), 1)
        return c_ < r_

    tieb0, tieb1 = _tieb(n), _tieb(_K0)

    aall = a0_ref[0]
    for p in range(1, a0_ref.shape[0]):
        aall = aall + a0_ref[p]

    xt0all = xt0_ref[...]
    x1s = [jnp.maximum(_dot(aall[g], xt0all[g * n:(g + 1) * n])
                       + b0_ref[...], 0.0) for g in range(_G)]

    s0s = [_sigmoid(_dot(x1s[g], w0col)) for g in range(_G)]
    s0rows = jnp.transpose(jnp.concatenate(s0s, axis=1))
    ms = [_before_mat(s0s[g], s0rows[g:g + 1], tieb0) for g in range(_G)]
    r0cs = [_dot(ms[g], ones128) for g in range(_G)]
    r0rows = jnp.transpose(jnp.concatenate(r0cs, axis=1))
    p0ts = [jnp.where(i0c == r0rows[g:g + 1], f1, f0) for g in range(_G)]
    p0s = [jnp.where(r0cs[g] == i0r, f1, f0) for g in range(_G)]
    t0s = [_dot00(s0s[g], p0s[g]) for g in range(_G)]
    nx0s = [_dot(p0ts[g] * s0rows[g:g + 1], x1s[g]) for g in range(_G)]
    pas = [_dot(p0ts[g], aall[g]) for g in range(_G)]
    a1s = [_dot(pas[g], p0s[g]) for g in range(_G)]
    h1s = [_dot(p0ts[g], rw11) for g in range(_G)]

    nx0all = jnp.concatenate(nx0s, axis=0)
    h1e = jnp.concatenate(
        [jnp.concatenate(h1s, axis=0),
         jnp.full((_G * _K0, 1), f1)], axis=1)
    hg1 = _dot(h1e, e9)
    y1all = _dot(nx0all, w2a1)
    xt1all = _dot(hg1 * y1all, rmat)
    x2s = [jnp.maximum(_dot(a1s[g], xt1all[g * _K0:(g + 1) * _K0])
                       + b1_ref[...], 0.0) for g in range(_G)]

    s1s = [_sigmoid(_dot(x2s[g], w1col)) for g in range(_G)]
    s1rows = jnp.transpose(jnp.concatenate(s1s, axis=1))
    m1s = [_before_mat(s1s[g], s1rows[g:g + 1], tieb1) for g in range(_G)]
    r1cs = [_dot(m1s[g], ones64) for g in range(_G)]
    r1rows = jnp.transpose(jnp.concatenate(r1cs, axis=1))
    p1ts = [jnp.where(i1c == r1rows[g:g + 1], f1, f0) for g in range(_G)]
    p1s = [jnp.where(r1cs[g] == i1r, f1, f0) for g in range(_G)]
    t1s = [_dot00(s1s[g], p1s[g]) for g in range(_G)]
    nx1s = [_dot(p1ts[g] * s1rows[g:g + 1], x2s[g]) for g in range(_G)]

    rows = [jnp.concatenate(
        [jnp.mean(nx0s[g], axis=0, keepdims=True),
         jnp.max(nx0s[g], axis=0, keepdims=True),
         jnp.mean(nx1s[g], axis=0, keepdims=True),
         jnp.max(nx1s[g], axis=0, keepdims=True)], axis=1)
        for g in range(_G)]

    t0all = jnp.concatenate(t0s, axis=0)
    t1all = jnp.concatenate(t1s, axis=0)

    r = jnp.concatenate(rows, axis=0)
    c0 = 1.0 / math.sqrt(1.0 + 1e-5)
    h = _dot(r, m1w_ref[...]) + m1b_ref[...]
    h = jnp.where(h >= 0, h, m1a_ref[...] * h) * c0 * m1g_ref[...] + m1be_ref[...]
    h = _dot(h, m2w_ref[...]) + m2b_ref[...]
    h = jnp.where(h >= 0, h, m2a_ref[...] * h) * c0 * m2g_ref[...] + m2be_ref[...]
    logits = _dot(h, m3w_ref[...]) + m3b_ref[...]
    mx = jnp.max(logits, axis=1, keepdims=True)
    e = jnp.exp(logits - mx)
    out_ref[...] = jnp.concatenate(
        [logits - mx - jnp.log(jnp.sum(e, axis=1, keepdims=True)),
         _softmax_row(t0all), _softmax_row(t1all), t0all], axis=1)


def _gnn_tc(x3, a0, w10, w2a0, b0, w0, w11, w2a1, b1, w1,
            m1w, m1b, m1a, m1g, m1be, m2w, m2b, m2a, m2g, m2be, m3w, m3b):
    bsz = x3.shape[0]
    grid = (bsz // _G,)

    def full(shape):
        zeros = (0,) * len(shape)
        return pl.BlockSpec(shape, lambda b, z=zeros: z)

    xt0 = pl.pallas_call(
        _tca_body,
        grid=grid,
        in_specs=[
            pl.BlockSpec((_G, _N_ROI, _N_ROI), lambda b: (b, 0, 0)),
            full(w10.shape), full(w2a0.shape),
        ],
        out_specs=pl.BlockSpec((_G * _N_ROI, 32), lambda b: (b, 0)),
        out_shape=jax.ShapeDtypeStruct((bsz * _N_ROI, 32), jnp.float32),
    )(x3, w10, w2a0)

    nc = a0.shape[0]
    in_specs = [
        pl.BlockSpec((_G * _N_ROI, 32), lambda b: (b, 0)),
        pl.BlockSpec((nc, _G, _N_ROI, _N_ROI), lambda b: (0, b, 0, 0)),
        full(b0.shape), full(w0.shape),
        full(w11.shape), full(w2a1.shape), full(b1.shape), full(w1.shape),
        full(m1w.shape), full(m1b.shape), full(m1a.shape), full(m1g.shape),
        full(m1be.shape), full(m2w.shape), full(m2b.shape), full(m2a.shape),
        full(m2g.shape), full(m2be.shape), full(m3w.shape), full(m3b.shape),
    ]
    npk = 2 + _K0 + _K1 + _K0
    out = pl.pallas_call(
        _tc_body,
        grid=grid,
        in_specs=in_specs,
        out_specs=pl.BlockSpec((_G, npk), lambda b: (b, 0)),
        out_shape=jax.ShapeDtypeStruct((bsz, npk), jnp.float32),
    )(xt0, a0, b0, w0, w11, w2a1, b1, w1,
      m1w, m1b, m1a, m1g, m1be, m2w, m2b, m2a, m2g, m2be, m3w, m3b)
    return (out[:, :2], out[:, 2:2 + _K0],
            out[:, 2 + _K0:2 + _K0 + _K1], out[:, 2 + _K0 + _K1:])


def _aug_weights(w2, b2, in_c):
    w2r = w2.reshape(_NCL, in_c, 32).transpose(1, 0, 2).reshape(in_c, _NCL * 32)
    return jnp.concatenate([w2r, b2.reshape(in_c, 32)], axis=1)


def kernel(x, edge_index, edge_attr, pos, batch, params):
    nroi = _N_ROI
    bsz = x.shape[0] // nroi
    e = edge_attr.shape[0]
    info = plsc.get_sparse_core_info()
    ns = int(info.num_subcores)
    nc = 1
    nw = nc * ns
    ch = e // nw
    n_a0 = bsz * nroi * nroi

    src = edge_index[0]
    dst = edge_index[1]
    a0p = _adj_builder(nc, ns, ch, n_a0)(
        dst.reshape(nw, ch), src.reshape(nw, ch),
        edge_attr.reshape(nw, ch),
        jnp.zeros((n_a0 // ns,), jnp.float32))
    a0 = a0p.reshape(nc, bsz, nroi, nroi)

    p = params
    w2a0 = _aug_weights(p['conv0_W2'], p['conv0_b2'], nroi)
    w2a1 = _aug_weights(p['conv1_W2'], p['conv1_b2'], 32)
    xo, sn0, sn1, sc0 = _gnn_tc(
        x.reshape(bsz, nroi, x.shape[1]), a0,
        p['conv0_W1'], w2a0, p['conv0_bias'].reshape(1, 32),
        p['pool0_w'].reshape(1, 32),
        p['conv1_W1'], w2a1, p['conv1_bias'].reshape(1, 32),
        p['pool1_w'].reshape(1, 32),
        p['mlp1_W'], p['mlp1_b'].reshape(1, -1), p['mlp1_a'].reshape(1, 1),
        p['mlp1_gamma'].reshape(1, -1), p['mlp1_beta'].reshape(1, -1),
        p['mlp2_W'], p['mlp2_b'].reshape(1, -1), p['mlp2_a'].reshape(1, 1),
        p['mlp2_gamma'].reshape(1, -1), p['mlp2_beta'].reshape(1, -1),
        p['mlp3_W'], p['mlp3_b'].reshape(1, -1))
    return (xo, p['pool0_w'], p['pool1_w'], sn0, sn1, sc0)
